# baseline (speedup 1.0000x reference)
import functools

import jax
import jax.numpy as jnp
from jax import lax
from jax.experimental import pallas as pl
from jax.experimental.pallas import tpu as pltpu
from jax.experimental.pallas import tpu_sc as plsc

N = 10000
E = 320000
D = 128
G = 64

NC = 2
NS = 16
NW = NC * NS

CH = 128
GS = 64
HALF = 40
EPT = 2 * HALF * CH
PAD = EPT - E // NW

NACC = 10240
RPS = NACC // NS
ZR = 128

BN = 1000
NB = N // BN


def _sc_gather_scatter_add(m, idx4):
    mesh = plsc.VectorSubcoreMesh(core_axis_name="c", subcore_axis_name="s",
                                  num_cores=NC, num_subcores=NS)

    @functools.partial(
        pl.kernel,
        out_type=jax.ShapeDtypeStruct((NC, NACC, D), jnp.float32),
        mesh=mesh,
        scratch_types=[
            pltpu.VMEM((2, HALF, CH), jnp.int32),
            pltpu.VMEM((2 * CH, D), jnp.float32),
            pltpu.VMEM_SHARED((NACC, D), jnp.float32),
            pltpu.SemaphoreType.DMA((4,)),
        ],
    )
    def k(m_hbm, idx_hbm, out_hbm, idxv, rows, acc, gsems):
        c = lax.axis_index("c")
        s = lax.axis_index("s")
        wid = c * NS + s
        @pl.loop(0, ZR)
        def _(i):
            for j16 in range(0, D, 16):
                rows[i, pl.ds(j16, 16)] = jnp.zeros((16,), jnp.float32)

        @pl.loop(0, RPS, step=ZR)
        def _(r):
            pltpu.sync_copy(rows.at[pl.ds(0, ZR), :],
                            acc.at[pl.ds(s * RPS + r, ZR), :])

        plsc.subcore_barrier()

        for h in range(2):
            pltpu.sync_copy(idx_hbm.at[wid, h], idxv)

            @pl.loop(0, 2)
            def _(j):
                for p in range(2):
                    pltpu.async_copy(
                        m_hbm.at[idxv.at[0, j, pl.ds(p * GS, GS)]],
                        rows.at[pl.ds((j & 1) * CH + p * GS, GS), :],
                        gsems.at[(j & 1) * 2 + p])

            @pl.loop(0, HALF)
            def _(j):
                off = (j & 1) * CH
                for p in range(2):
                    pltpu.make_async_copy(
                        m_hbm.at[idxv.at[0, j, pl.ds(p * GS, GS)]],
                        rows.at[pl.ds(off + p * GS, GS), :],
                        gsems.at[(j & 1) * 2 + p]).wait()
                pltpu.sync_copy(rows.at[pl.ds(off, CH), :],
                                acc.at[idxv.at[1, j]], add=True)

                @pl.when(j + 2 < HALF)
                def _():
                    for p in range(2):
                        pltpu.async_copy(
                            m_hbm.at[idxv.at[0, j + 2, pl.ds(p * GS, GS)]],
                            rows.at[pl.ds(off + p * GS, GS), :],
                            gsems.at[(j & 1) * 2 + p])

        plsc.subcore_barrier()
        pltpu.sync_copy(acc.at[pl.ds(s * RPS, RPS), :],
                        out_hbm.at[c, pl.ds(s * RPS, RPS), :])

    return k(m, idx4)


def _tc_matmul(h, w):
    def body(h_ref, w_ref, o_ref):
        o_ref[...] = jnp.dot(h_ref[...], w_ref[...],
                             preferred_element_type=jnp.float32)

    return pl.pallas_call(
        body,
        grid=(NB,),
        in_specs=[pl.BlockSpec((BN, D), lambda i: (i, 0)),
                  pl.BlockSpec((D, D), lambda i: (0, 0))],
        out_specs=pl.BlockSpec((BN, D), lambda i: (i, 0)),
        out_shape=jax.ShapeDtypeStruct((N, D), jnp.float32),
    )(h, w)


def _gru_block(aggp_ref, h_ref, wi_ref, wh_ref, bi_ref, bh_ref):
    agg = aggp_ref[0] + aggp_ref[1]
    h = h_ref[...]
    gi = jnp.dot(agg, wi_ref[...], preferred_element_type=jnp.float32) + bi_ref[...]
    gh = jnp.dot(h, wh_ref[...], preferred_element_type=jnp.float32) + bh_ref[...]
    r = jax.nn.sigmoid(gi[:, :D] + gh[:, :D])
    z = jax.nn.sigmoid(gi[:, D:2 * D] + gh[:, D:2 * D])
    n = jnp.tanh(gi[:, 2 * D:] + r * gh[:, 2 * D:])
    return (1.0 - z) * n + z * h


def _tc_gru_next(aggp, h, wi_t, wh_t, bi, bh, w_next):
    def body(aggp_ref, h_ref, wi_ref, wh_ref, bi_ref, bh_ref, wn_ref,
             h_out, m_out):
        hn = _gru_block(aggp_ref, h_ref, wi_ref, wh_ref, bi_ref, bh_ref)
        h_out[...] = hn
        m_out[...] = jnp.dot(hn, wn_ref[...], preferred_element_type=jnp.float32)

    return pl.pallas_call(
        body,
        grid=(NB,),
        in_specs=[pl.BlockSpec((NC, BN, D), lambda i: (0, i, 0)),
                  pl.BlockSpec((BN, D), lambda i: (i, 0)),
                  pl.BlockSpec((D, 3 * D), lambda i: (0, 0)),
                  pl.BlockSpec((D, 3 * D), lambda i: (0, 0)),
                  pl.BlockSpec((1, 3 * D), lambda i: (0, 0)),
                  pl.BlockSpec((1, 3 * D), lambda i: (0, 0)),
                  pl.BlockSpec((D, D), lambda i: (0, 0))],
        out_specs=[pl.BlockSpec((BN, D), lambda i: (i, 0)),
                   pl.BlockSpec((BN, D), lambda i: (i, 0))],
        out_shape=[jax.ShapeDtypeStruct((N, D), jnp.float32),
                   jax.ShapeDtypeStruct((N, D), jnp.float32)],
    )(aggp, h, wi_t, wh_t, bi, bh, w_next)


def _tc_gru_pool(aggp, h, wi_t, wh_t, bi, bh, batch3):
    def body(aggp_ref, h_ref, wi_ref, wh_ref, bi_ref, bh_ref, b_ref,
             out_ref, sums, cnts):
        i = pl.program_id(0)

        @pl.when(i == 0)
        def _():
            sums[...] = jnp.zeros_like(sums)
            cnts[...] = jnp.zeros_like(cnts)

        hn = _gru_block(aggp_ref, h_ref, wi_ref, wh_ref, bi_ref, bh_ref)
        hr = jnp.maximum(hn, 0.0)
        ids = b_ref[...].reshape(1, BN)
        oh = (lax.broadcasted_iota(jnp.int32, (G, BN), 0) == ids)
        oh = oh.astype(jnp.float32)
        sums[...] += jnp.dot(oh, hr, preferred_element_type=jnp.float32)
        cnts[...] += jnp.dot(oh, jnp.ones((BN, D), jnp.float32),
                             preferred_element_type=jnp.float32)

        @pl.when(i == NB - 1)
        def _():
            out_ref[...] = sums[...] / jnp.maximum(cnts[...], 1.0)

    return pl.pallas_call(
        body,
        grid=(NB,),
        in_specs=[pl.BlockSpec((NC, BN, D), lambda i: (0, i, 0)),
                  pl.BlockSpec((BN, D), lambda i: (i, 0)),
                  pl.BlockSpec((D, 3 * D), lambda i: (0, 0)),
                  pl.BlockSpec((D, 3 * D), lambda i: (0, 0)),
                  pl.BlockSpec((1, 3 * D), lambda i: (0, 0)),
                  pl.BlockSpec((1, 3 * D), lambda i: (0, 0)),
                  pl.BlockSpec((1, 1, BN), lambda i: (i, 0, 0))],
        out_specs=pl.BlockSpec((G, D), lambda i: (0, 0)),
        out_shape=jax.ShapeDtypeStruct((G, D), jnp.float32),
        scratch_shapes=[pltpu.VMEM((G, D), jnp.float32),
                        pltpu.VMEM((G, D), jnp.float32)],
    )(aggp, h, wi_t, wh_t, bi, bh, batch3)


def kernel(x, edge_index, batch, weight, w_ih, w_hh, b_ih, b_hh):
    src, dst = lax.sort_key_val(edge_index[0], edge_index[1])
    srcp = jnp.pad(src.reshape(NW, E // NW), ((0, 0), (0, PAD)))
    dstp = jnp.pad(dst.reshape(NW, E // NW), ((0, 0), (0, PAD)),
                   constant_values=N)
    idx4 = jnp.stack([srcp.reshape(NW, 2, HALF, CH),
                      dstp.reshape(NW, 2, HALF, CH)], axis=2)
    wi_t = w_ih.T.reshape(D, 3 * D)
    wh_t = w_hh.T.reshape(D, 3 * D)
    bi = b_ih.reshape(1, 3 * D)
    bh = b_hh.reshape(1, 3 * D)
    batch3 = batch.reshape(NB, 1, BN)

    m0 = _tc_matmul(x, weight[0])
    agg0 = _sc_gather_scatter_add(m0, idx4)
    h1, m1 = _tc_gru_next(agg0, x, wi_t, wh_t, bi, bh, weight[1])
    agg1 = _sc_gather_scatter_add(m1, idx4)
    return _tc_gru_pool(agg1, h1, wi_t, wh_t, bi, bh, batch3)


# speedup vs baseline: 1.6388x; 1.6388x over previous
import functools

import jax
import jax.numpy as jnp
from jax import lax
from jax.experimental import pallas as pl
from jax.experimental.pallas import tpu as pltpu
from jax.experimental.pallas import tpu_sc as plsc

N = 10000
E = 320000
D = 128
G = 64

NC = 2
NS = 16
NW = NC * NS

CH = 128
HALF = 40
EPT = 2 * HALF * CH
PAD = EPT - E // NW

NACC = 10240
RPS = NACC // NS
ZR = 128

BN = 1000
NB = N // BN


def _sc_gather_scatter_add(m, idx5):
    mesh = plsc.VectorSubcoreMesh(core_axis_name="c", subcore_axis_name="s",
                                  num_cores=NC, num_subcores=NS)

    @functools.partial(
        pl.kernel,
        out_type=jax.ShapeDtypeStruct((NC, NACC, D), jnp.float32),
        mesh=mesh,
        scratch_types=[
            pltpu.VMEM((2, HALF, CH), jnp.int32),
            pltpu.VMEM((2 * CH, D), jnp.float32),
            pltpu.VMEM((32, D), jnp.float32),
            pltpu.VMEM_SHARED((NACC, D), jnp.float32),
            pltpu.SemaphoreType.DMA((2,)),
        ],
    )
    def k(m_hbm, idx_hbm, out_hbm, idxv, rows, zb, acc, gsems):
        c = lax.axis_index("c")
        s = lax.axis_index("s")
        wid = c * NS + s
        pltpu.sync_copy(idx_hbm.at[wid, 0], idxv)

        @pl.loop(0, 2)
        def _(j):
            pltpu.async_copy(m_hbm.at[idxv.at[0, j]],
                             rows.at[pl.ds((j & 1) * CH, CH), :],
                             gsems.at[j & 1])

        @pl.loop(0, 32)
        def _(i):
            for j16 in range(0, D, 16):
                zb[i, pl.ds(j16, 16)] = jnp.zeros((16,), jnp.float32)

        @pl.loop(0, RPS, step=32)
        def _(r):
            pltpu.sync_copy(zb, acc.at[pl.ds(s * RPS + r, 32), :])

        plsc.subcore_barrier()

        for h in range(2):
            if h > 0:
                pltpu.sync_copy(idx_hbm.at[wid, 1], idxv)

                @pl.loop(0, 2)
                def _(j):
                    pltpu.async_copy(m_hbm.at[idxv.at[0, j]],
                                     rows.at[pl.ds((j & 1) * CH, CH), :],
                                     gsems.at[j & 1])

            @pl.loop(0, HALF)
            def _(j):
                off = (j & 1) * CH
                pltpu.make_async_copy(m_hbm.at[idxv.at[0, j]],
                                      rows.at[pl.ds(off, CH), :],
                                      gsems.at[j & 1]).wait()
                pltpu.sync_copy(rows.at[pl.ds(off, CH), :],
                                acc.at[idxv.at[1, j]], add=True)

                @pl.when(j + 2 < HALF)
                def _():
                    pltpu.async_copy(m_hbm.at[idxv.at[0, j + 2]],
                                     rows.at[pl.ds(off, CH), :],
                                     gsems.at[j & 1])

        plsc.subcore_barrier()
        pltpu.sync_copy(acc.at[pl.ds(s * RPS, RPS), :],
                        out_hbm.at[c, pl.ds(s * RPS, RPS), :])

    return k(m, idx5)


def _tc_matmul(h, w):
    def body(h_ref, w_ref, o_ref):
        o_ref[...] = jnp.dot(h_ref[...], w_ref[...],
                             preferred_element_type=jnp.float32)

    return pl.pallas_call(
        body,
        grid=(NB,),
        in_specs=[pl.BlockSpec((BN, D), lambda i: (i, 0)),
                  pl.BlockSpec((D, D), lambda i: (0, 0))],
        out_specs=pl.BlockSpec((BN, D), lambda i: (i, 0)),
        out_shape=jax.ShapeDtypeStruct((N, D), jnp.float32),
    )(h, w)


def _gru_block(aggp_ref, h_ref, wi_ref, wh_ref, bi_ref, bh_ref):
    agg = aggp_ref[0] + aggp_ref[1]
    h = h_ref[...]
    gi = jnp.dot(agg, wi_ref[...], preferred_element_type=jnp.float32) + bi_ref[...]
    gh = jnp.dot(h, wh_ref[...], preferred_element_type=jnp.float32) + bh_ref[...]
    r = jax.nn.sigmoid(gi[:, :D] + gh[:, :D])
    z = jax.nn.sigmoid(gi[:, D:2 * D] + gh[:, D:2 * D])
    n = jnp.tanh(gi[:, 2 * D:] + r * gh[:, 2 * D:])
    return (1.0 - z) * n + z * h


def _tc_gru_next(aggp, h, wi_t, wh_t, bi, bh, w_next):
    def body(aggp_ref, h_ref, wi_ref, wh_ref, bi_ref, bh_ref, wn_ref,
             h_out, m_out):
        hn = _gru_block(aggp_ref, h_ref, wi_ref, wh_ref, bi_ref, bh_ref)
        h_out[...] = hn
        m_out[...] = jnp.dot(hn, wn_ref[...], preferred_element_type=jnp.float32)

    return pl.pallas_call(
        body,
        grid=(NB,),
        in_specs=[pl.BlockSpec((NC, BN, D), lambda i: (0, i, 0)),
                  pl.BlockSpec((BN, D), lambda i: (i, 0)),
                  pl.BlockSpec((D, 3 * D), lambda i: (0, 0)),
                  pl.BlockSpec((D, 3 * D), lambda i: (0, 0)),
                  pl.BlockSpec((1, 3 * D), lambda i: (0, 0)),
                  pl.BlockSpec((1, 3 * D), lambda i: (0, 0)),
                  pl.BlockSpec((D, D), lambda i: (0, 0))],
        out_specs=[pl.BlockSpec((BN, D), lambda i: (i, 0)),
                   pl.BlockSpec((BN, D), lambda i: (i, 0))],
        out_shape=[jax.ShapeDtypeStruct((N, D), jnp.float32),
                   jax.ShapeDtypeStruct((N, D), jnp.float32)],
    )(aggp, h, wi_t, wh_t, bi, bh, w_next)


def _tc_gru_pool(aggp, h, wi_t, wh_t, bi, bh, batch3):
    def body(aggp_ref, h_ref, wi_ref, wh_ref, bi_ref, bh_ref, b_ref,
             out_ref, sums, cnts):
        i = pl.program_id(0)

        @pl.when(i == 0)
        def _():
            sums[...] = jnp.zeros_like(sums)
            cnts[...] = jnp.zeros_like(cnts)

        hn = _gru_block(aggp_ref, h_ref, wi_ref, wh_ref, bi_ref, bh_ref)
        hr = jnp.maximum(hn, 0.0)
        ids = b_ref[...].reshape(1, BN)
        oh = (lax.broadcasted_iota(jnp.int32, (G, BN), 0) == ids)
        oh = oh.astype(jnp.float32)
        sums[...] += jnp.dot(oh, hr, preferred_element_type=jnp.float32)
        cnts[...] += jnp.dot(oh, jnp.ones((BN, D), jnp.float32),
                             preferred_element_type=jnp.float32)

        @pl.when(i == NB - 1)
        def _():
            out_ref[...] = sums[...] / jnp.maximum(cnts[...], 1.0)

    return pl.pallas_call(
        body,
        grid=(NB,),
        in_specs=[pl.BlockSpec((NC, BN, D), lambda i: (0, i, 0)),
                  pl.BlockSpec((BN, D), lambda i: (i, 0)),
                  pl.BlockSpec((D, 3 * D), lambda i: (0, 0)),
                  pl.BlockSpec((D, 3 * D), lambda i: (0, 0)),
                  pl.BlockSpec((1, 3 * D), lambda i: (0, 0)),
                  pl.BlockSpec((1, 3 * D), lambda i: (0, 0)),
                  pl.BlockSpec((1, 1, BN), lambda i: (i, 0, 0))],
        out_specs=pl.BlockSpec((G, D), lambda i: (0, 0)),
        out_shape=jax.ShapeDtypeStruct((G, D), jnp.float32),
        scratch_shapes=[pltpu.VMEM((G, D), jnp.float32),
                        pltpu.VMEM((G, D), jnp.float32)],
    )(aggp, h, wi_t, wh_t, bi, bh, batch3)


def kernel(x, edge_index, batch, weight, w_ih, w_hh, b_ih, b_hh):
    srcp = jnp.pad(edge_index[0].reshape(NW, E // NW), ((0, 0), (0, PAD)))
    dstp = jnp.pad(edge_index[1].reshape(NW, E // NW), ((0, 0), (0, PAD)),
                   constant_values=N)
    idx5 = jnp.stack([srcp.reshape(NW, 2, HALF, CH),
                      dstp.reshape(NW, 2, HALF, CH)], axis=2)
    wi_t = w_ih.T.reshape(D, 3 * D)
    wh_t = w_hh.T.reshape(D, 3 * D)
    bi = b_ih.reshape(1, 3 * D)
    bh = b_hh.reshape(1, 3 * D)
    batch3 = batch.reshape(NB, 1, BN)

    m0 = _tc_matmul(x, weight[0])
    agg0 = _sc_gather_scatter_add(m0, idx5)
    h1, m1 = _tc_gru_next(agg0, x, wi_t, wh_t, bi, bh, weight[1])
    agg1 = _sc_gather_scatter_add(m1, idx5)
    return _tc_gru_pool(agg1, h1, wi_t, wh_t, bi, bh, batch3)
